# Initial kernel scaffold; baseline (speedup 1.0000x reference)
import jax
import jax.numpy as jnp
from jax.experimental import pallas as pl


def kernel(x, edge_index, edge_weight, W1, b1, W2, b2):
    raise NotImplementedError("write your pallas kernel here")


# speedup vs baseline: 3.8356x; 3.8356x over previous
import dataclasses
import functools

import jax
import jax.numpy as jnp
from jax import lax
from jax.experimental import pallas as pl
from jax.experimental.pallas import tpu as pltpu
from jax.experimental.pallas import tpu_sc as plsc

N = 10000
NP = 10240
E = 320000
C = 128

NC = 2
NS = 16
NT = NC * NS
CPT = C // NT
EPT = E // NT
EC = 4000
NCHUNK = E // EC

F32 = jnp.float32
I32 = jnp.int32

_MESH = plsc.VectorSubcoreMesh(core_axis_name="c", subcore_axis_name="s",
                               num_cores=NC, num_subcores=NS)

_SC_PARAMS = pltpu.CompilerParams()
if "needs_layout_passes" in pltpu.CompilerParams.__dataclass_fields__:
    _SC_PARAMS = dataclasses.replace(_SC_PARAMS, needs_layout_passes=False)


def _wid():
    return lax.axis_index("s") * NC + lax.axis_index("c")


@functools.partial(
    pl.kernel,
    out_type=jax.ShapeDtypeStruct((NT, NP), F32),
    mesh=_MESH,
    compiler_params=_SC_PARAMS,
    scratch_types=[
        pltpu.VMEM((NP,), F32),
        pltpu.VMEM((EPT,), I32),
        pltpu.VMEM((EPT,), F32),
    ],
)
def _deg_kernel(pk_hbm, ew_hbm, degp_hbm, degbuf, pkbuf, ewbuf):
    tid = _wid()
    base = tid * EPT

    @pl.loop(0, NP, step=16)
    def _zero(i):
        degbuf[pl.ds(i, 16)] = jnp.zeros((16,), F32)

    pltpu.sync_copy(pk_hbm.at[pl.ds(base, EPT)], pkbuf)
    pltpu.sync_copy(ew_hbm.at[pl.ds(base, EPT)], ewbuf)

    @pl.loop(0, EPT, step=16)
    def _acc(i):
        pk = pkbuf[pl.ds(i, 16)]
        srcv = jnp.bitwise_and(pk, 65535)
        wv = ewbuf[pl.ds(i, 16)]
        plsc.addupdate_scatter(degbuf, [srcv], wv)

    pltpu.sync_copy(degbuf, degp_hbm.at[tid])


def _prep_body(degp_ref, x_ref, dis_ref, xT_ref):
    deg = jnp.sum(degp_ref[...], axis=0)
    pos = deg > 0.0
    safe = jnp.where(pos, deg, 1.0)
    dis = jnp.where(pos, lax.rsqrt(safe), 0.0)
    dis_ref[...] = dis[None, :]
    xT_ref[:, pl.ds(0, N)] = x_ref[...].T
    xT_ref[:, pl.ds(N, NP - N)] = jnp.zeros((C, NP - N), F32)


_prep = pl.pallas_call(
    _prep_body,
    out_shape=[
        jax.ShapeDtypeStruct((1, NP), F32),
        jax.ShapeDtypeStruct((C, NP), F32),
    ],
)


@functools.partial(
    pl.kernel,
    out_type=jax.ShapeDtypeStruct((E,), F32),
    mesh=_MESH,
    compiler_params=_SC_PARAMS,
    scratch_types=[
        pltpu.VMEM((NP,), F32),
        pltpu.VMEM((EPT,), I32),
        pltpu.VMEM((EPT,), F32),
        pltpu.VMEM((EPT,), F32),
    ],
)
def _edgew_kernel(pk_hbm, ew_hbm, dis_hbm, wn_hbm, disbuf, pkbuf, ewbuf, wnbuf):
    tid = _wid()
    base = tid * EPT
    pltpu.sync_copy(dis_hbm.at[0], disbuf)
    pltpu.sync_copy(pk_hbm.at[pl.ds(base, EPT)], pkbuf)
    pltpu.sync_copy(ew_hbm.at[pl.ds(base, EPT)], ewbuf)

    @pl.loop(0, EPT, step=16)
    def _go(i):
        pk = pkbuf[pl.ds(i, 16)]
        srcv = jnp.bitwise_and(pk, 65535)
        dstv = lax.shift_right_logical(pk, 16)
        a = plsc.load_gather(disbuf, [srcv])
        b = plsc.load_gather(disbuf, [dstv])
        wv = ewbuf[pl.ds(i, 16)]
        wnbuf[pl.ds(i, 16)] = -(a * wv * b)

    pltpu.sync_copy(wnbuf, wn_hbm.at[pl.ds(base, EPT)])


@functools.partial(
    pl.kernel,
    out_type=jax.ShapeDtypeStruct((C, NP), F32),
    mesh=_MESH,
    compiler_params=_SC_PARAMS,
    scratch_types=[
        pltpu.VMEM((NP,), F32),
        pltpu.VMEM((NP,), F32),
        pltpu.VMEM((NP,), F32),
        pltpu.VMEM((NP,), F32),
        pltpu.VMEM((NP,), F32),
        pltpu.VMEM((NP,), F32),
        pltpu.VMEM((NP,), F32),
        pltpu.VMEM((NP,), F32),
        pltpu.VMEM((EC,), I32),
        pltpu.VMEM((EC,), I32),
        pltpu.VMEM((EC,), F32),
        pltpu.VMEM((EC,), F32),
        pltpu.SemaphoreType.DMA,
        pltpu.SemaphoreType.DMA,
        pltpu.SemaphoreType.DMA,
        pltpu.SemaphoreType.DMA,
    ],
)
def _mv_kernel(hT_hbm, pk_hbm, wn_hbm, outT_hbm,
               h0, h1, h2, h3, o0, o1, o2, o3,
               pkb0, pkb1, wb0, wb1, spk0, spk1, sw0, sw1):
    tid = _wid()
    row = tid * CPT
    hb = (h0, h1, h2, h3)
    ob = (o0, o1, o2, o3)
    pkb = (pkb0, pkb1)
    wb = (wb0, wb1)
    spk = (spk0, spk1)
    sw = (sw0, sw1)

    for c in range(CPT):
        pltpu.sync_copy(hT_hbm.at[row + c], hb[c])

    @pl.loop(0, NP, step=16)
    def _zero(i):
        z = jnp.zeros((16,), F32)
        for c in range(CPT):
            ob[c][pl.ds(i, 16)] = z

    def _issue(b, ci):
        pltpu.async_copy(pk_hbm.at[pl.ds(ci * EC, EC)], pkb[b], spk[b])
        pltpu.async_copy(wn_hbm.at[pl.ds(ci * EC, EC)], wb[b], sw[b])

    def _wait(b):
        pltpu.make_async_copy(pk_hbm.at[pl.ds(0, EC)], pkb[b], spk[b]).wait()
        pltpu.make_async_copy(wn_hbm.at[pl.ds(0, EC)], wb[b], sw[b]).wait()

    def _process(b):
        @pl.loop(0, EC, step=16)
        def _grp(i):
            pk = pkb[b][pl.ds(i, 16)]
            srcv = jnp.bitwise_and(pk, 65535)
            dstv = lax.shift_right_logical(pk, 16)
            wv = wb[b][pl.ds(i, 16)]
            for c in range(CPT):
                hv = plsc.load_gather(hb[c], [srcv])
                plsc.addupdate_scatter(ob[c], [dstv], hv * wv)

    _issue(0, 0)

    @pl.loop(0, NCHUNK // 2)
    def _pair(k):
        _issue(1, 2 * k + 1)
        _wait(0)
        _process(0)

        @pl.when(k < NCHUNK // 2 - 1)
        def _():
            _issue(0, 2 * k + 2)

        _wait(1)
        _process(1)

    for c in range(CPT):
        pltpu.sync_copy(ob[c], outT_hbm.at[row + c])


def _dotT(A, B):
    return lax.dot_general(
        A, B, (((0,), (0,)), ((), ())),
        precision=lax.Precision.HIGHEST,
        preferred_element_type=F32,
    )


def _layer1_body(xT_ref, t1_ref, m2_ref, W_ref, b_ref, hT_ref):
    W0, W1_, W2_ = W_ref[0], W_ref[1], W_ref[2]
    acc = _dotT(W0 - W2_, xT_ref[...])
    acc = acc + _dotT(W1_, t1_ref[...])
    acc = acc + _dotT(2.0 * W2_, m2_ref[...])
    acc = acc + b_ref[0][:, None]
    hT_ref[...] = jnp.maximum(acc, 0.0)


_layer1 = pl.pallas_call(
    _layer1_body, out_shape=jax.ShapeDtypeStruct((C, NP), F32))


def _layer2_body(hT_ref, t1_ref, m2_ref, W_ref, b_ref, out_ref):
    W0, W1_, W2_ = W_ref[0], W_ref[1], W_ref[2]
    acc = _dotT(W0 - W2_, hT_ref[...])
    acc = acc + _dotT(W1_, t1_ref[...])
    acc = acc + _dotT(2.0 * W2_, m2_ref[...])
    acc = acc + b_ref[0][:, None]
    out_ref[...] = acc[:, :N].T


_layer2 = pl.pallas_call(
    _layer2_body, out_shape=jax.ShapeDtypeStruct((N, C), F32))


def kernel(x, edge_index, edge_weight, W1, b1, W2, b2):
    src = edge_index[0].astype(I32)
    dst = edge_index[1].astype(I32)
    packed = jnp.bitwise_or(src, jnp.left_shift(dst, 16))
    ew = edge_weight.astype(F32)

    degp = _deg_kernel(packed, ew)
    dis, xT = _prep(degp, x)
    wn = _edgew_kernel(packed, ew, dis)

    t1T = _mv_kernel(xT, packed, wn)
    m2T = _mv_kernel(t1T, packed, wn)
    hT = _layer1(xT, t1T, m2T, W1, b1.reshape(1, C))
    t1T2 = _mv_kernel(hT, packed, wn)
    m2T2 = _mv_kernel(t1T2, packed, wn)
    return _layer2(hT, t1T2, m2T2, W2, b2.reshape(1, C))
